# baseline (speedup 1.0000x reference)
import functools

import jax
import jax.numpy as jnp
from jax import lax
from jax.experimental import pallas as pl
from jax.experimental.pallas import tpu as pltpu
from jax.experimental.pallas import tpu_sc as plsc

_CH = 96


def _make_gather(R, V, C, NW):
    NCH = R // _CH
    mesh = plsc.VectorSubcoreMesh(core_axis_name="c", subcore_axis_name="s")

    @functools.partial(
        pl.kernel,
        mesh=mesh,
        out_type=jax.ShapeDtypeStruct((R, C), jnp.float32),
        scratch_types=[
            pltpu.VMEM((_CH,), jnp.int32),
            pltpu.VMEM((_CH, C), jnp.float32),
            pltpu.SemaphoreType.DMA,
        ],
    )
    def k(x_hbm, idx_hbm, out_hbm, idx_v, rows_v, sem):
        wid = lax.axis_index("s") * 2 + lax.axis_index("c")
        n_w = (NCH - wid + NW - 1) // NW

        def body(i, carry):
            base = (wid + i * NW) * _CH
            base = pl.multiple_of(base, 8)
            pltpu.sync_copy(idx_hbm.at[pl.ds(base, _CH)], idx_v)
            pltpu.async_copy(x_hbm.at[idx_v], rows_v, sem).wait()
            pltpu.sync_copy(rows_v, out_hbm.at[pl.ds(base, _CH)])
            return carry

        lax.fori_loop(0, n_w, body, 0)

    return k


def kernel(x, center_idx):
    B, N_IN, C = x.shape
    N_OUT = center_idx.shape[0]
    R = B * N_OUT

    x_flat = x.reshape(B * N_IN, C)
    offs = (jnp.arange(B, dtype=jnp.int32) * N_IN)[:, None]
    full_idx = (center_idx[None, :].astype(jnp.int32) + offs).reshape(-1)

    out = _make_gather(R, B * N_IN, C, 32)(x_flat, full_idx)
    return out.reshape(B, N_OUT, C)


# speedup vs baseline: 1.3152x; 1.3152x over previous
import functools

import jax
import jax.numpy as jnp
from jax import lax
from jax.experimental import pallas as pl
from jax.experimental.pallas import tpu as pltpu
from jax.experimental.pallas import tpu_sc as plsc

_CH = 96
_NBUF = 4
_NW = 32


def _make_gather(R, C):
    NCH = R // _CH
    base_n = NCH // _NW
    rem = NCH - base_n * _NW
    max_n = base_n + (1 if rem else 0)
    slab = max_n * _CH
    mesh = plsc.VectorSubcoreMesh(core_axis_name="c", subcore_axis_name="s")

    @functools.partial(
        pl.kernel,
        mesh=mesh,
        out_type=jax.ShapeDtypeStruct((R, C), jnp.float32),
        scratch_types=[
            pltpu.VMEM((slab,), jnp.int32),
            pltpu.VMEM((_NBUF, _CH, C), jnp.float32),
        ]
        + [pltpu.SemaphoreType.DMA] * (2 * _NBUF),
    )
    def k(x_hbm, idx_hbm, out_hbm, idx_v, rows_v, *sems):
        gsem = sems[:_NBUF]
        ssem = sems[_NBUF:]
        wid = lax.axis_index("s") * 2 + lax.axis_index("c")
        n_w = base_n + (wid < rem).astype(jnp.int32)
        s_w = wid * base_n + jnp.minimum(wid, rem)

        pltpu.sync_copy(idx_hbm.at[pl.ds(s_w * _CH, slab)], idx_v)

        def group(g, carry):
            for b in range(_NBUF):
                l = g * _NBUF + b

                @pl.when(jnp.logical_and(l < n_w, l >= _NBUF))
                def _():
                    pltpu.make_async_copy(
                        rows_v.at[b], out_hbm.at[pl.ds(0, _CH)], ssem[b]
                    ).wait()

                @pl.when(l < n_w)
                def _():
                    pltpu.async_copy(
                        x_hbm.at[idx_v.at[pl.ds(l * _CH, _CH)]],
                        rows_v.at[b],
                        gsem[b],
                    )

            for b in range(_NBUF):
                l = g * _NBUF + b

                @pl.when(l < n_w)
                def _():
                    pltpu.make_async_copy(
                        x_hbm.at[idx_v.at[pl.ds(l * _CH, _CH)]],
                        rows_v.at[b],
                        gsem[b],
                    ).wait()
                    pltpu.async_copy(
                        rows_v.at[b],
                        out_hbm.at[pl.ds((s_w + l) * _CH, _CH)],
                        ssem[b],
                    )

            return carry

        lax.fori_loop(0, (n_w + _NBUF - 1) // _NBUF, group, 0)

        for b in range(_NBUF):
            pltpu.make_async_copy(
                rows_v.at[b], out_hbm.at[pl.ds(0, _CH)], ssem[b]
            ).wait()

    return k


def kernel(x, center_idx):
    B, N_IN, C = x.shape
    N_OUT = center_idx.shape[0]
    R = B * N_OUT
    NCH = R // _CH
    max_n = NCH // _NW + (1 if NCH % _NW else 0)

    x_flat = x.reshape(B * N_IN, C)
    offs = (jnp.arange(B, dtype=jnp.int32) * N_IN)[:, None]
    full_idx = (center_idx[None, :].astype(jnp.int32) + offs).reshape(-1)
    pad = _NW * max_n * _CH - R
    if pad:
        full_idx = jnp.concatenate([full_idx, jnp.zeros((pad,), jnp.int32)])

    out = _make_gather(R, C)(x_flat, full_idx)
    return out.reshape(B, N_OUT, C)


# speedup vs baseline: 1.3180x; 1.0021x over previous
import functools

import jax
import jax.numpy as jnp
from jax import lax
from jax.experimental import pallas as pl
from jax.experimental.pallas import tpu as pltpu
from jax.experimental.pallas import tpu_sc as plsc

_CH = 96
_NBUF = 8
_NW = 32


def _make_gather(R, C):
    NCH = R // _CH
    base_n = NCH // _NW
    rem = NCH - base_n * _NW
    max_n = base_n + (1 if rem else 0)
    slab = max_n * _CH
    mesh = plsc.VectorSubcoreMesh(core_axis_name="c", subcore_axis_name="s")

    @functools.partial(
        pl.kernel,
        mesh=mesh,
        out_type=jax.ShapeDtypeStruct((R, C), jnp.float32),
        scratch_types=[
            pltpu.VMEM((slab,), jnp.int32),
            pltpu.VMEM((_NBUF, _CH, C), jnp.float32),
        ]
        + [pltpu.SemaphoreType.DMA] * (2 * _NBUF),
    )
    def k(x_hbm, idx_hbm, out_hbm, idx_v, rows_v, *sems):
        gsem = sems[:_NBUF]
        ssem = sems[_NBUF:]
        wid = lax.axis_index("s") * 2 + lax.axis_index("c")
        n_w = base_n + (wid < rem).astype(jnp.int32)
        s_w = wid * base_n + jnp.minimum(wid, rem)

        pltpu.sync_copy(idx_hbm.at[pl.ds(s_w * _CH, slab)], idx_v)

        def group(g, carry):
            for b in range(_NBUF):
                l = g * _NBUF + b

                @pl.when(jnp.logical_and(l < n_w, l >= _NBUF))
                def _():
                    pltpu.make_async_copy(
                        rows_v.at[b], out_hbm.at[pl.ds(0, _CH)], ssem[b]
                    ).wait()

                @pl.when(l < n_w)
                def _():
                    pltpu.async_copy(
                        x_hbm.at[idx_v.at[pl.ds(l * _CH, _CH)]],
                        rows_v.at[b],
                        gsem[b],
                    )

            for b in range(_NBUF):
                l = g * _NBUF + b

                @pl.when(l < n_w)
                def _():
                    pltpu.make_async_copy(
                        x_hbm.at[idx_v.at[pl.ds(l * _CH, _CH)]],
                        rows_v.at[b],
                        gsem[b],
                    ).wait()
                    pltpu.async_copy(
                        rows_v.at[b],
                        out_hbm.at[pl.ds((s_w + l) * _CH, _CH)],
                        ssem[b],
                    )

            return carry

        lax.fori_loop(0, (n_w + _NBUF - 1) // _NBUF, group, 0)

        for b in range(_NBUF):
            pltpu.make_async_copy(
                rows_v.at[b], out_hbm.at[pl.ds(0, _CH)], ssem[b]
            ).wait()

    return k


def kernel(x, center_idx):
    B, N_IN, C = x.shape
    N_OUT = center_idx.shape[0]
    R = B * N_OUT
    NCH = R // _CH
    max_n = NCH // _NW + (1 if NCH % _NW else 0)

    x_flat = x.reshape(B * N_IN, C)
    offs = (jnp.arange(B, dtype=jnp.int32) * N_IN)[:, None]
    full_idx = (center_idx[None, :].astype(jnp.int32) + offs).reshape(-1)
    pad = _NW * max_n * _CH - R
    if pad:
        full_idx = jnp.concatenate([full_idx, jnp.zeros((pad,), jnp.int32)])

    out = _make_gather(R, C)(x_flat, full_idx)
    return out.reshape(B, N_OUT, C)


# speedup vs baseline: 2.0668x; 1.5681x over previous
import functools

import jax
import jax.numpy as jnp
from jax import lax
from jax.experimental import pallas as pl
from jax.experimental.pallas import tpu as pltpu
from jax.experimental.pallas import tpu_sc as plsc

_CH = 96
_NBUF = 8
_NW = 32


def _make_gather(B, N_IN, N_OUT, C, tail_pad):
    n_full = N_OUT // _CH
    tail = N_OUT - n_full * _CH
    h0 = (n_full + 1) // 2
    h1 = n_full - h0
    slab = (max(h0, h1) * _CH + tail_pad + 7) // 8 * 8
    mesh = plsc.VectorSubcoreMesh(core_axis_name="c", subcore_axis_name="s")

    @functools.partial(
        pl.kernel,
        mesh=mesh,
        out_type=jax.ShapeDtypeStruct((B, N_OUT, C), jnp.float32),
        scratch_types=[
            pltpu.VMEM((slab,), jnp.int32),
            pltpu.VMEM((_NBUF, _CH, C), jnp.float32),
            pltpu.VMEM((max(tail_pad, 8), C), jnp.float32),
            pltpu.VMEM((max(tail_pad, 8),), jnp.int32),
        ]
        + [pltpu.SemaphoreType.DMA] * (2 * _NBUF + 1),
    )
    def k(x_hbm, idx_hbm, tidx_hbm, out_hbm, idx_v, rows_v, tail_v, tidx_v, *sems):
        gsem = sems[:_NBUF]
        ssem = sems[_NBUF : 2 * _NBUF]
        tsem = sems[2 * _NBUF]
        wid = lax.axis_index("s") * 2 + lax.axis_index("c")
        b = wid // 2
        half = wid % 2
        n_w = jnp.where(half == 0, h0, h1)
        row0 = half * (h0 * _CH)

        pltpu.sync_copy(idx_hbm.at[pl.ds(half * (h0 * _CH), slab)], idx_v)

        def group(g, carry):
            for s in range(_NBUF):
                l = g * _NBUF + s

                @pl.when(jnp.logical_and(l < n_w, l >= _NBUF))
                def _():
                    pltpu.make_async_copy(
                        rows_v.at[s], out_hbm.at[b].at[pl.ds(0, _CH)], ssem[s]
                    ).wait()

                @pl.when(l < n_w)
                def _():
                    pltpu.async_copy(
                        x_hbm.at[b].at[idx_v.at[pl.ds(l * _CH, _CH)]],
                        rows_v.at[s],
                        gsem[s],
                    )

            for s in range(_NBUF):
                l = g * _NBUF + s

                @pl.when(l < n_w)
                def _():
                    pltpu.make_async_copy(
                        x_hbm.at[b].at[idx_v.at[pl.ds(l * _CH, _CH)]],
                        rows_v.at[s],
                        gsem[s],
                    ).wait()
                    pltpu.async_copy(
                        rows_v.at[s],
                        out_hbm.at[b].at[pl.ds(row0 + l * _CH, _CH)],
                        ssem[s],
                    )

            return carry

        lax.fori_loop(0, (n_w + _NBUF - 1) // _NBUF, group, 0)

        for s in range(_NBUF):
            pltpu.make_async_copy(
                rows_v.at[s], out_hbm.at[b].at[pl.ds(0, _CH)], ssem[s]
            ).wait()

        if tail:

            @pl.when(half == 1)
            def _():
                pltpu.sync_copy(tidx_hbm, tidx_v)
                pltpu.async_copy(
                    x_hbm.at[b].at[idx_v.at[pl.ds(h1 * _CH, tail_pad)]],
                    tail_v,
                    tsem,
                ).wait()
                pltpu.async_copy(tail_v, out_hbm.at[b].at[tidx_v], tsem).wait()

    return k


def kernel(x, center_idx):
    B, N_IN, C = x.shape
    N_OUT = center_idx.shape[0]
    n_full = N_OUT // _CH
    tail = N_OUT - n_full * _CH
    tail_pad = (tail + 7) // 8 * 8
    h0 = (n_full + 1) // 2
    h1 = n_full - h0
    slab = (max(h0, h1) * _CH + tail_pad + 7) // 8 * 8
    idx_pad = h0 * _CH + slab

    idx = center_idx.astype(jnp.int32)
    if idx_pad > N_OUT:
        idx = jnp.concatenate([idx, jnp.zeros((idx_pad - N_OUT,), jnp.int32)])
    tidx = n_full * _CH + jnp.arange(max(tail_pad, 8), dtype=jnp.int32)

    return _make_gather(B, N_IN, N_OUT, C, tail_pad)(x, idx, tidx)


# speedup vs baseline: 4.3667x; 2.1128x over previous
import functools

import jax
import jax.numpy as jnp
from jax import lax
from jax.experimental import pallas as pl
from jax.experimental.pallas import tpu as pltpu
from jax.experimental.pallas import tpu_sc as plsc

_K = 16
_NBUF = 3
_NW = 32


def _make_gather(N_IN, N_OUT, B, C):
    rows_w = N_OUT // _NW
    rem = N_OUT - rows_w * _NW
    n_full = rows_w // _K
    ctail = rows_w - n_full * _K
    slab = (rows_w + rem + 7) // 8 * 8
    mesh = plsc.VectorSubcoreMesh(core_axis_name="c", subcore_axis_name="s")

    @functools.partial(
        pl.kernel,
        mesh=mesh,
        out_type=jax.ShapeDtypeStruct((N_OUT, B, C), jnp.float32),
        scratch_types=[
            pltpu.VMEM((slab,), jnp.int32),
            pltpu.VMEM((_NBUF, _K, B, C), jnp.float32),
        ]
        + [pltpu.SemaphoreType.DMA] * (2 * _NBUF + 1),
    )
    def k(x_hbm, idx_hbm, out_hbm, idx_v, rows_v, *sems):
        gsem = sems[:_NBUF]
        ssem = sems[_NBUF : 2 * _NBUF]
        xsem = sems[2 * _NBUF]
        wid = lax.axis_index("s") * 2 + lax.axis_index("c")
        row0 = wid * rows_w

        pltpu.sync_copy(idx_hbm.at[pl.ds(row0, slab)], idx_v)

        def group(g, carry):
            for s in range(_NBUF):
                l = g * _NBUF + s

                @pl.when(jnp.logical_and(l < n_full, l >= _NBUF))
                def _():
                    pltpu.make_async_copy(
                        rows_v.at[s], out_hbm.at[pl.ds(0, _K)], ssem[s]
                    ).wait()

                @pl.when(l < n_full)
                def _():
                    pltpu.async_copy(
                        x_hbm.at[idx_v.at[pl.ds(l * _K, _K)]],
                        rows_v.at[s],
                        gsem[s],
                    )

            for s in range(_NBUF):
                l = g * _NBUF + s

                @pl.when(l < n_full)
                def _():
                    pltpu.make_async_copy(
                        x_hbm.at[idx_v.at[pl.ds(l * _K, _K)]],
                        rows_v.at[s],
                        gsem[s],
                    ).wait()
                    pltpu.async_copy(
                        rows_v.at[s],
                        out_hbm.at[pl.ds(row0 + l * _K, _K)],
                        ssem[s],
                    )

            return carry

        lax.fori_loop(0, (n_full + _NBUF - 1) // _NBUF, group, 0)

        for s in range(_NBUF):
            pltpu.make_async_copy(
                rows_v.at[s], out_hbm.at[pl.ds(0, _K)], ssem[s]
            ).wait()

        if ctail:

            def _():
                pltpu.async_copy(
                    x_hbm.at[idx_v.at[pl.ds(n_full * _K, ctail)]],
                    rows_v.at[0].at[pl.ds(0, ctail)],
                    xsem,
                ).wait()
                pltpu.async_copy(
                    rows_v.at[0].at[pl.ds(0, ctail)],
                    out_hbm.at[pl.ds(row0 + n_full * _K, ctail)],
                    xsem,
                ).wait()

            _()

        if rem:

            @pl.when(wid == _NW - 1)
            def _():
                pltpu.async_copy(
                    x_hbm.at[idx_v.at[pl.ds(rows_w, rem)]],
                    rows_v.at[0].at[pl.ds(0, rem)],
                    xsem,
                ).wait()
                pltpu.async_copy(
                    rows_v.at[0].at[pl.ds(0, rem)],
                    out_hbm.at[pl.ds(row0 + rows_w, rem)],
                    xsem,
                ).wait()

    return k


def kernel(x, center_idx):
    B, N_IN, C = x.shape
    N_OUT = center_idx.shape[0]
    rows_w = N_OUT // _NW
    rem = N_OUT - rows_w * _NW
    slab = (rows_w + rem + 7) // 8 * 8
    idx_len = (_NW - 1) * rows_w + slab

    idx = center_idx.astype(jnp.int32)
    if idx_len > N_OUT:
        idx = jnp.concatenate([idx, jnp.zeros((idx_len - N_OUT,), jnp.int32)])

    x_t = jnp.transpose(x, (1, 0, 2))
    out_t = _make_gather(N_IN, N_OUT, B, C)(x_t, idx)
    return jnp.transpose(out_t, (1, 0, 2))


# speedup vs baseline: 4.4659x; 1.0227x over previous
import functools

import jax
import jax.numpy as jnp
from jax import lax
from jax.experimental import pallas as pl
from jax.experimental.pallas import tpu as pltpu
from jax.experimental.pallas import tpu_sc as plsc

_K = 8
_SET = 3
_NW = 32


def _make_gather(N_IN, N_OUT, B, C):
    rows_w = N_OUT // _NW
    rem = N_OUT - rows_w * _NW
    n_full = rows_w // _K
    ctail = rows_w - n_full * _K
    slab = (rows_w + rem + 7) // 8 * 8
    n_groups = (n_full + _SET - 1) // _SET
    n_super = (n_groups + 3 + 1) // 2
    mesh = plsc.VectorSubcoreMesh(core_axis_name="c", subcore_axis_name="s")

    @functools.partial(
        pl.kernel,
        mesh=mesh,
        out_type=jax.ShapeDtypeStruct((N_OUT, B, C), jnp.float32),
        scratch_types=[
            pltpu.VMEM((slab,), jnp.int32),
            pltpu.VMEM((2 * _SET, _K, B, C), jnp.float32),
        ]
        + [pltpu.SemaphoreType.DMA] * (4 * _SET + 1),
    )
    def k(x_hbm, idx_hbm, out_hbm, idx_v, rows_v, *sems):
        gsem = sems[: 2 * _SET]
        ssem = sems[2 * _SET : 4 * _SET]
        xsem = sems[4 * _SET]
        wid = lax.axis_index("s") * 2 + lax.axis_index("c")
        row0 = wid * rows_w

        pltpu.sync_copy(idx_hbm.at[pl.ds(row0, slab)], idx_v)

        def run_group(g, sb):
            osb = _SET - sb
            for i in range(_SET):
                s = sb + i
                f = (g - 2) * _SET + i

                @pl.when(jnp.logical_and(f >= 0, f < n_full))
                def _():
                    pltpu.make_async_copy(
                        rows_v.at[s], out_hbm.at[pl.ds(0, _K)], ssem[s]
                    ).wait()

                l = g * _SET + i

                @pl.when(l < n_full)
                def _():
                    pltpu.async_copy(
                        x_hbm.at[idx_v.at[pl.ds(l * _K, _K)]],
                        rows_v.at[s],
                        gsem[s],
                    )

            for i in range(_SET):
                s = osb + i
                p = (g - 1) * _SET + i

                @pl.when(jnp.logical_and(p >= 0, p < n_full))
                def _():
                    pltpu.make_async_copy(
                        x_hbm.at[idx_v.at[pl.ds(0, _K)]],
                        rows_v.at[s],
                        gsem[s],
                    ).wait()
                    pltpu.async_copy(
                        rows_v.at[s],
                        out_hbm.at[pl.ds(row0 + p * _K, _K)],
                        ssem[s],
                    )

        def super_group(h, carry):
            run_group(2 * h, 0)
            run_group(2 * h + 1, _SET)
            return carry

        lax.fori_loop(0, n_super, super_group, 0)

        if ctail:

            def _():
                pltpu.async_copy(
                    x_hbm.at[idx_v.at[pl.ds(n_full * _K, ctail)]],
                    rows_v.at[0].at[pl.ds(0, ctail)],
                    xsem,
                ).wait()
                pltpu.async_copy(
                    rows_v.at[0].at[pl.ds(0, ctail)],
                    out_hbm.at[pl.ds(row0 + n_full * _K, ctail)],
                    xsem,
                ).wait()

            _()

        if rem:

            @pl.when(wid == _NW - 1)
            def _():
                pltpu.async_copy(
                    x_hbm.at[idx_v.at[pl.ds(rows_w, rem)]],
                    rows_v.at[0].at[pl.ds(0, rem)],
                    xsem,
                ).wait()
                pltpu.async_copy(
                    rows_v.at[0].at[pl.ds(0, rem)],
                    out_hbm.at[pl.ds(row0 + rows_w, rem)],
                    xsem,
                ).wait()

    return k


def kernel(x, center_idx):
    B, N_IN, C = x.shape
    N_OUT = center_idx.shape[0]
    rows_w = N_OUT // _NW
    rem = N_OUT - rows_w * _NW
    slab = (rows_w + rem + 7) // 8 * 8
    idx_len = (_NW - 1) * rows_w + slab

    idx = center_idx.astype(jnp.int32)
    if idx_len > N_OUT:
        idx = jnp.concatenate([idx, jnp.zeros((idx_len - N_OUT,), jnp.int32)])

    x_t = jnp.transpose(x, (1, 0, 2))
    out_t = _make_gather(N_IN, N_OUT, B, C)(x_t, idx)
    return jnp.transpose(out_t, (1, 0, 2))


# speedup vs baseline: 4.5020x; 1.0081x over previous
import functools

import jax
import jax.numpy as jnp
from jax import lax
from jax.experimental import pallas as pl
from jax.experimental.pallas import tpu as pltpu
from jax.experimental.pallas import tpu_sc as plsc

_K = 16
_SET = 1
_NW = 32


def _make_gather(N_IN, N_OUT, B, C):
    rows_w = N_OUT // _NW
    rem = N_OUT - rows_w * _NW
    n_full = rows_w // _K
    ctail = rows_w - n_full * _K
    slab = (rows_w + rem + 7) // 8 * 8
    n_groups = (n_full + _SET - 1) // _SET
    n_super = (n_groups + 3 + 1) // 2
    mesh = plsc.VectorSubcoreMesh(core_axis_name="c", subcore_axis_name="s")

    @functools.partial(
        pl.kernel,
        mesh=mesh,
        out_type=jax.ShapeDtypeStruct((N_OUT, B, C), jnp.float32),
        scratch_types=[
            pltpu.VMEM((slab,), jnp.int32),
            pltpu.VMEM((2 * _SET, _K, B, C), jnp.float32),
        ]
        + [pltpu.SemaphoreType.DMA] * (4 * _SET + 1),
    )
    def k(x_hbm, idx_hbm, out_hbm, idx_v, rows_v, *sems):
        gsem = sems[: 2 * _SET]
        ssem = sems[2 * _SET : 4 * _SET]
        xsem = sems[4 * _SET]
        wid = lax.axis_index("s") * 2 + lax.axis_index("c")
        row0 = wid * rows_w

        pltpu.sync_copy(idx_hbm.at[pl.ds(row0, slab)], idx_v)

        def run_group(g, sb):
            osb = _SET - sb
            for i in range(_SET):
                s = sb + i
                f = (g - 2) * _SET + i

                @pl.when(jnp.logical_and(f >= 0, f < n_full))
                def _():
                    pltpu.make_async_copy(
                        rows_v.at[s], out_hbm.at[pl.ds(0, _K)], ssem[s]
                    ).wait()

                l = g * _SET + i

                @pl.when(l < n_full)
                def _():
                    pltpu.async_copy(
                        x_hbm.at[idx_v.at[pl.ds(l * _K, _K)]],
                        rows_v.at[s],
                        gsem[s],
                    )

            for i in range(_SET):
                s = osb + i
                p = (g - 1) * _SET + i

                @pl.when(jnp.logical_and(p >= 0, p < n_full))
                def _():
                    pltpu.make_async_copy(
                        x_hbm.at[idx_v.at[pl.ds(0, _K)]],
                        rows_v.at[s],
                        gsem[s],
                    ).wait()
                    pltpu.async_copy(
                        rows_v.at[s],
                        out_hbm.at[pl.ds(row0 + p * _K, _K)],
                        ssem[s],
                    )

        def super_group(h, carry):
            run_group(2 * h, 0)
            run_group(2 * h + 1, _SET)
            return carry

        lax.fori_loop(0, n_super, super_group, 0)

        if ctail:

            def _():
                pltpu.async_copy(
                    x_hbm.at[idx_v.at[pl.ds(n_full * _K, ctail)]],
                    rows_v.at[0].at[pl.ds(0, ctail)],
                    xsem,
                ).wait()
                pltpu.async_copy(
                    rows_v.at[0].at[pl.ds(0, ctail)],
                    out_hbm.at[pl.ds(row0 + n_full * _K, ctail)],
                    xsem,
                ).wait()

            _()

        if rem:

            @pl.when(wid == _NW - 1)
            def _():
                pltpu.async_copy(
                    x_hbm.at[idx_v.at[pl.ds(rows_w, rem)]],
                    rows_v.at[0].at[pl.ds(0, rem)],
                    xsem,
                ).wait()
                pltpu.async_copy(
                    rows_v.at[0].at[pl.ds(0, rem)],
                    out_hbm.at[pl.ds(row0 + rows_w, rem)],
                    xsem,
                ).wait()

    return k


def kernel(x, center_idx):
    B, N_IN, C = x.shape
    N_OUT = center_idx.shape[0]
    rows_w = N_OUT // _NW
    rem = N_OUT - rows_w * _NW
    slab = (rows_w + rem + 7) // 8 * 8
    idx_len = (_NW - 1) * rows_w + slab

    idx = center_idx.astype(jnp.int32)
    if idx_len > N_OUT:
        idx = jnp.concatenate([idx, jnp.zeros((idx_len - N_OUT,), jnp.int32)])

    x_t = jnp.transpose(x, (1, 0, 2))
    out_t = _make_gather(N_IN, N_OUT, B, C)(x_t, idx)
    return jnp.transpose(out_t, (1, 0, 2))
